# baseline (speedup 1.0000x reference)
import functools

import jax
import jax.numpy as jnp
from jax import lax
from jax.experimental import pallas as pl
from jax.experimental.pallas import tpu as pltpu
from jax.experimental.pallas import tpu_sc as plsc

_N = 10000
_NROW = 10240
_E = 160000
_NSUB = 16
_K = 128
_NB = 80
_EPAD = _NSUB * _NB * _K
_C = 64
_NCH = 512 // _C
_ROWS_PER_TILE = _NROW // _NSUB

_f32 = jnp.float32
_i32 = jnp.int32

_MESH = plsc.VectorSubcoreMesh(core_axis_name="c", subcore_axis_name="s")
_SC_PARAMS = pltpu.CompilerParams(use_tc_tiling_on_sc=False)


def _deg_kernel_body(srcp, dstp, ones_hbm, zrow_hbm, out_hbm,
                     sidx, didx, ones_v, zbuf, acc_s, acc_d):
    c = lax.axis_index("c")
    s = lax.axis_index("s")
    hb = _NB // 2
    rpt = _ROWS_PER_TILE
    pltpu.sync_copy(ones_hbm, ones_v)
    pltpu.sync_copy(zrow_hbm, zbuf)
    pltpu.sync_copy(zbuf, acc_s.at[pl.ds(s * rpt, rpt)])
    pltpu.sync_copy(zbuf, acc_d.at[pl.ds(s * rpt, rpt)])
    plsc.subcore_barrier()
    pltpu.sync_copy(srcp.at[s].at[pl.ds(c * hb, hb)], sidx)
    pltpu.sync_copy(dstp.at[s].at[pl.ds(c * hb, hb)], didx)

    @pl.loop(0, hb)
    def _(b):
        pltpu.sync_copy(ones_v, acc_s.at[sidx.at[b]], add=True)
        pltpu.sync_copy(ones_v, acc_d.at[didx.at[b]], add=True)

    plsc.subcore_barrier()
    pltpu.sync_copy(acc_s.at[pl.ds(s * rpt, rpt)],
                    out_hbm.at[c].at[0].at[pl.ds(s * rpt, rpt)])
    pltpu.sync_copy(acc_d.at[pl.ds(s * rpt, rpt)],
                    out_hbm.at[c].at[1].at[pl.ds(s * rpt, rpt)])


_deg_kernel = functools.partial(
    pl.kernel,
    out_type=jax.ShapeDtypeStruct((2, 2, _NROW, 16), _f32),
    mesh=_MESH,
    compiler_params=_SC_PARAMS,
    scratch_types=[
        pltpu.VMEM((_NB // 2, _K), _i32),
        pltpu.VMEM((_NB // 2, _K), _i32),
        pltpu.VMEM((_K, 16), _f32),
        pltpu.VMEM((_ROWS_PER_TILE, 16), _f32),
        pltpu.VMEM_SHARED((_NROW, 16), _f32),
        pltpu.VMEM_SHARED((_NROW, 16), _f32),
    ],
)(_deg_kernel_body)


def _agg512_body(p_hbm, srcp, dstp, zrow_hbm, out_hbm,
                 sidx, didx, buf0, buf1, acc, sg0, sg1, ss0, ss1,
                 *, do_gather=True, do_scatter=True):
    c = lax.axis_index("c")
    s = lax.axis_index("s")
    rpt = _ROWS_PER_TILE
    pltpu.sync_copy(srcp.at[s], sidx)
    pltpu.sync_copy(dstp.at[s], didx)

    for ci in range(_NCH // 2):
        chunk = c * _NCH // 2 + ci
        tbl = p_hbm.at[chunk]
        och = out_hbm.at[chunk]

        pltpu.sync_copy(zrow_hbm, buf0)
        for j in range(rpt // _K):
            pltpu.sync_copy(buf0, acc.at[pl.ds(s * rpt + j * _K, _K)])
        plsc.subcore_barrier()

        def gstart(b, buf, sem):
            if do_gather:
                pltpu.async_copy(tbl.at[sidx.at[b]], buf, sem)

        def gwait(buf, sem):
            if do_gather:
                pltpu.make_async_copy(tbl.at[sidx.at[0]], buf, sem).wait()

        def sstart(b, buf, sem):
            if do_scatter:
                pltpu.async_copy(buf, acc.at[didx.at[b]], sem, add=True)

        def swait(buf, sem):
            if do_scatter:
                pltpu.make_async_copy(buf, acc.at[didx.at[0]], sem).wait()

        gstart(0, buf0, sg0)

        @pl.loop(0, _NB, step=2)
        def _(b):
            gwait(buf0, sg0)

            @pl.when(b > 0)
            def _():
                swait(buf1, ss1)

            gstart(b + 1, buf1, sg1)
            sstart(b, buf0, ss0)
            gwait(buf1, sg1)
            swait(buf0, ss0)

            @pl.when(b + 2 < _NB)
            def _():
                gstart(b + 2, buf0, sg0)

            sstart(b + 1, buf1, ss1)

        swait(buf1, ss1)
        plsc.subcore_barrier()
        pltpu.sync_copy(acc.at[pl.ds(s * rpt, rpt)], och.at[pl.ds(s * rpt, rpt)])
        plsc.subcore_barrier()


def _gdiag_body(p_hbm, srcp, out_hbm, sidx, buf0, buf1, sg0, sg1, *, ncch):
    c = lax.axis_index("c")
    s = lax.axis_index("s")
    pltpu.sync_copy(srcp.at[s], sidx)
    for ci in range(ncch):
        chunk = c * ncch + ci
        tbl = p_hbm.at[chunk]

        def gstart(b, buf, sem):
            pltpu.async_copy(tbl.at[sidx.at[b]], buf, sem)

        def gwait(buf, sem):
            pltpu.make_async_copy(tbl.at[sidx.at[0]], buf, sem).wait()

        gstart(0, buf0, sg0)

        @pl.loop(0, _NB, step=2)
        def _(b):
            gwait(buf0, sg0)
            gstart(b + 1, buf1, sg1)
            gwait(buf1, sg1)

            @pl.when(b + 2 < _NB)
            def _():
                gstart(b + 2, buf0, sg0)

    pltpu.sync_copy(buf0, out_hbm.at[c * _NSUB + s])


def _make_gdiag(ncch, w, dt, tiled):
    return functools.partial(
        pl.kernel,
        out_type=jax.ShapeDtypeStruct((32, _K, w), dt),
        mesh=_MESH,
        compiler_params=None if tiled else _SC_PARAMS,
        scratch_types=[
            pltpu.VMEM((_NB, _K), _i32),
            pltpu.VMEM((_K, w), dt),
            pltpu.VMEM((_K, w), dt),
            pltpu.SemaphoreType.DMA,
            pltpu.SemaphoreType.DMA,
        ],
    )(functools.partial(_gdiag_body, ncch=ncch))


_gdiag_bf16_64 = _make_gdiag(4, 64, jnp.bfloat16, False)
_gdiag_f32_128 = _make_gdiag(2, 128, _f32, False)
_gdiag_f32_128t = _make_gdiag(2, 128, _f32, True)


def _make_agg512(**kw):
    return functools.partial(
        pl.kernel,
        out_type=jax.ShapeDtypeStruct((_NCH, _NROW, _C), _f32),
        mesh=_MESH,
        compiler_params=_SC_PARAMS,
        scratch_types=[
            pltpu.VMEM((_NB, _K), _i32),
            pltpu.VMEM((_NB, _K), _i32),
            pltpu.VMEM((_K, _C), _f32),
            pltpu.VMEM((_K, _C), _f32),
            pltpu.VMEM_SHARED((_NROW, _C), _f32),
            pltpu.SemaphoreType.DMA,
            pltpu.SemaphoreType.DMA,
            pltpu.SemaphoreType.DMA,
            pltpu.SemaphoreType.DMA,
        ],
    )(functools.partial(_agg512_body, **kw))


_agg512_gonly = _make_agg512(do_scatter=False)
_agg512_sonly = _make_agg512(do_gather=False)

_agg512 = functools.partial(
    pl.kernel,
    out_type=jax.ShapeDtypeStruct((_NCH, _NROW, _C), _f32),
    mesh=_MESH,
    compiler_params=_SC_PARAMS,
    scratch_types=[
        pltpu.VMEM((_NB, _K), _i32),
        pltpu.VMEM((_NB, _K), _i32),
        pltpu.VMEM((_K, _C), _f32),
        pltpu.VMEM((_K, _C), _f32),
        pltpu.VMEM_SHARED((_NROW, _C), _f32),
        pltpu.SemaphoreType.DMA,
        pltpu.SemaphoreType.DMA,
        pltpu.SemaphoreType.DMA,
        pltpu.SemaphoreType.DMA,
    ],
)(_agg512_body)


def _agg16_body(p_hbm, srcp, dstp, zrow_hbm, out_hbm,
                sidx, didx, buf0, buf1, acc, sg0, sg1, ss0, ss1):
    c = lax.axis_index("c")
    s = lax.axis_index("s")
    rpt = _ROWS_PER_TILE
    hb = _NB // 2
    pltpu.sync_copy(srcp.at[s].at[pl.ds(c * hb, hb)], sidx)
    pltpu.sync_copy(dstp.at[s].at[pl.ds(c * hb, hb)], didx)

    pltpu.sync_copy(zrow_hbm, buf0)
    for j in range(rpt // _K):
        pltpu.sync_copy(buf0, acc.at[pl.ds(s * rpt + j * _K, _K)])
    plsc.subcore_barrier()

    def gstart(b, buf, sem):
        pltpu.async_copy(p_hbm.at[sidx.at[b]], buf, sem)

    def gwait(buf, sem):
        pltpu.make_async_copy(p_hbm.at[sidx.at[0]], buf, sem).wait()

    def sstart(b, buf, sem):
        pltpu.async_copy(buf, acc.at[didx.at[b]], sem, add=True)

    def swait(buf, sem):
        pltpu.make_async_copy(buf, acc.at[didx.at[0]], sem).wait()

    gstart(0, buf0, sg0)

    @pl.loop(0, hb, step=2)
    def _(b):
        gwait(buf0, sg0)

        @pl.when(b > 0)
        def _():
            swait(buf1, ss1)

        gstart(b + 1, buf1, sg1)
        sstart(b, buf0, ss0)
        gwait(buf1, sg1)
        swait(buf0, ss0)

        @pl.when(b + 2 < hb)
        def _():
            gstart(b + 2, buf0, sg0)

        sstart(b + 1, buf1, ss1)

    swait(buf1, ss1)
    plsc.subcore_barrier()
    pltpu.sync_copy(acc.at[pl.ds(s * rpt, rpt)],
                    out_hbm.at[c].at[pl.ds(s * rpt, rpt)])


_agg16 = functools.partial(
    pl.kernel,
    out_type=jax.ShapeDtypeStruct((2, _NROW, 16), _f32),
    mesh=_MESH,
    compiler_params=_SC_PARAMS,
    scratch_types=[
        pltpu.VMEM((_NB // 2, _K), _i32),
        pltpu.VMEM((_NB // 2, _K), _i32),
        pltpu.VMEM((_K, 16), _f32),
        pltpu.VMEM((_K, 16), _f32),
        pltpu.VMEM_SHARED((_NROW, 16), _f32),
        pltpu.SemaphoreType.DMA,
        pltpu.SemaphoreType.DMA,
        pltpu.SemaphoreType.DMA,
        pltpu.SemaphoreType.DMA,
    ],
)(_agg16_body)


def _dense1_body(x_ref, w_ref, b_ref, o_ref):
    a = x_ref[0].astype(jnp.bfloat16)
    w = w_ref[0].astype(jnp.bfloat16)
    acc = lax.dot_general(a, w, (((1,), (0,)), ((), ())),
                          preferred_element_type=_f32)
    o_ref[...] = acc + b_ref[0]


def _dense1(xs, ws, bs):
    return pl.pallas_call(
        _dense1_body,
        grid=(2, 5),
        in_specs=[
            pl.BlockSpec((1, 1000, 256), lambda j, i: (j, i, 0)),
            pl.BlockSpec((1, 256, 512), lambda j, i: (j, 0, 0)),
            pl.BlockSpec((1, 1, 512), lambda j, i: (j, 0, 0)),
        ],
        out_specs=pl.BlockSpec((1000, 512), lambda j, i: (j * 5 + i, 0)),
        out_shape=jax.ShapeDtypeStruct((_NROW, 512), _f32),
    )(xs, ws, bs)


def _prep_body(h_ref, dg_ref, p1_ref, nin_ref, nout_ref):
    dg = dg_ref[...]
    d_out = dg[0, 0, :, 0:1] + dg[1, 0, :, 0:1]
    d_in = dg[0, 1, :, 0:1] + dg[1, 1, :, 0:1]
    no = jnp.where(d_out > 0, lax.rsqrt(d_out), 0.0)
    ni = jnp.where(d_in > 0, lax.rsqrt(d_in), 0.0)
    blk = no.shape[0]
    nout_ref[...] = jnp.broadcast_to(no, (blk, _C))
    nin_ref[...] = jnp.broadcast_to(ni, (blk, _C))
    hb = h_ref[...] * no
    for cc in range(_NCH):
        p1_ref[cc] = hb[:, cc * _C:(cc + 1) * _C]


def _prep(h, degp):
    blk = 1024
    return pl.pallas_call(
        _prep_body,
        grid=(_NROW // blk,),
        in_specs=[
            pl.BlockSpec((blk, 512), lambda i: (i, 0)),
            pl.BlockSpec((2, 2, blk, 16), lambda i: (0, 0, i, 0)),
        ],
        out_specs=[
            pl.BlockSpec((_NCH, blk, _C), lambda i: (0, i, 0)),
            pl.BlockSpec((blk, _C), lambda i: (i, 0)),
            pl.BlockSpec((blk, _C), lambda i: (i, 0)),
        ],
        out_shape=[
            jax.ShapeDtypeStruct((_NCH, _NROW, _C), _f32),
            jax.ShapeDtypeStruct((_NROW, _C), _f32),
            jax.ShapeDtypeStruct((_NROW, _C), _f32),
        ],
    )(h, degp)


def _mid_body(a_ref, nin_ref, nout_ref, b_ref, w_ref, o_ref, *, nwo):
    nin = nin_ref[...]
    nout = nout_ref[...]
    blk = nin.shape[0]
    acc = jnp.zeros((blk, nwo), _f32)
    for cc in range(_NCH):
        hc = jnp.maximum(a_ref[cc] * nin + b_ref[cc], 0.0) * nout
        acc = acc + lax.dot_general(
            hc.astype(jnp.bfloat16), w_ref[cc].astype(jnp.bfloat16),
            (((1,), (0,)), ((), ())), preferred_element_type=_f32)
    if nwo == 512:
        for cc in range(_NCH):
            o_ref[cc] = acc[:, cc * _C:(cc + 1) * _C]
    else:
        o_ref[...] = acc


def _mid(a, nin, nout, b, w):
    blk = 1024
    nwo = w.shape[2]
    if nwo == 512:
        out_spec = pl.BlockSpec((_NCH, blk, _C), lambda i: (0, i, 0))
        out_shape = jax.ShapeDtypeStruct((_NCH, _NROW, _C), _f32)
    else:
        out_spec = pl.BlockSpec((blk, nwo), lambda i: (i, 0))
        out_shape = jax.ShapeDtypeStruct((_NROW, nwo), _f32)
    return pl.pallas_call(
        functools.partial(_mid_body, nwo=nwo),
        grid=(_NROW // blk,),
        in_specs=[
            pl.BlockSpec((_NCH, blk, _C), lambda i: (0, i, 0)),
            pl.BlockSpec((blk, _C), lambda i: (i, 0)),
            pl.BlockSpec((blk, _C), lambda i: (i, 0)),
            pl.BlockSpec((_NCH, 1, _C), lambda i: (0, 0, 0)),
            pl.BlockSpec((_NCH, _C, nwo), lambda i: (0, 0, 0)),
        ],
        out_specs=out_spec,
        out_shape=out_shape,
    )(a, nin, nout, b, w)


def _final_body(a_ref, nin_ref, b_ref, o_ref):
    a = a_ref[0] + a_ref[1]
    o_ref[...] = a * nin_ref[:, :16] + b_ref[...]


def _final(a3p, nin, b2):
    blk = 1000
    return pl.pallas_call(
        _final_body,
        grid=(_N // blk,),
        in_specs=[
            pl.BlockSpec((2, blk, 16), lambda i: (0, i, 0)),
            pl.BlockSpec((blk, _C), lambda i: (i, 0)),
            pl.BlockSpec((1, 16), lambda i: (0, 0)),
        ],
        out_specs=pl.BlockSpec((blk, 16), lambda i: (i, 0)),
        out_shape=jax.ShapeDtypeStruct((_N, 16), _f32),
    )(a3p, nin, b2)


def kernel(x0, x1, edge_index, fc0_w, fc0_b, fc1_w, fc1_b, conv0_b, conv1_w,
           conv1_b, conv2_w, conv2_b):
    src = edge_index[0]
    dst = edge_index[1]
    pad = jnp.full((_EPAD - _E,), _N, _i32)
    srcp = jnp.concatenate([src, pad]).reshape(_NSUB, _NB, _K)
    dstp = jnp.concatenate([dst, pad]).reshape(_NSUB, _NB, _K)

    ones128 = jnp.ones((_K, 16), _f32)
    zrow1 = jnp.zeros((_ROWS_PER_TILE, 16), _f32)
    zrow128 = jnp.zeros((_K, _C), _f32)
    zrow16 = jnp.zeros((_K, 16), _f32)

    xs = jnp.stack([x0, x1])
    ws = jnp.stack([fc0_w, fc1_w])
    bs = jnp.stack([fc0_b, fc1_b]).reshape(2, 1, 512)

    degp = _deg_kernel(srcp, dstp, ones128, zrow1)
    h = _dense1(xs, ws, bs)
    p1, nin, nout = _prep(h, degp)

    a1 = _agg512(p1, srcp, dstp, zrow128)
    d1 = _gdiag_bf16_64(p1.astype(jnp.bfloat16), srcp)
    pw = jnp.reshape(p1, (4, _NROW, 128))
    d2 = _gdiag_f32_128(pw, srcp)
    d3 = _gdiag_f32_128t(pw, srcp)
    a1 = a1 + 0.0 * (d1.astype(_f32).sum() + d2.sum() + d3.sum())
    p2 = _mid(a1, nin, nout, conv0_b.reshape(_NCH, 1, _C),
              conv1_w.reshape(_NCH, _C, 512))
    a2 = _agg512(p2, srcp, dstp, zrow128)
    p3 = _mid(a2, nin, nout, conv1_b.reshape(_NCH, 1, _C),
              conv2_w.reshape(_NCH, _C, 16))
    a3p = _agg16(p3, srcp, dstp, zrow16)
    out = _final(a3p, nin, conv2_b.reshape(1, 16))
    return out


# speedup vs baseline: 1.2611x; 1.2611x over previous
import functools

import jax
import jax.numpy as jnp
from jax import lax
from jax.experimental import pallas as pl
from jax.experimental.pallas import tpu as pltpu
from jax.experimental.pallas import tpu_sc as plsc

_N = 10000
_NROW = 10240
_E = 160000
_NSUB = 16
_K = 128
_NB = 80
_EPAD = _NSUB * _NB * _K
_C = 64
_NCH = 512 // _C
_ROWS_PER_TILE = _NROW // _NSUB

_f32 = jnp.float32
_i32 = jnp.int32

_MESH = plsc.VectorSubcoreMesh(core_axis_name="c", subcore_axis_name="s")
_SC_PARAMS = pltpu.CompilerParams(use_tc_tiling_on_sc=False)


def _deg_kernel_body(srcp, dstp, ones_hbm, zrow_hbm, out_hbm,
                     sidx, didx, ones_v, zbuf, acc_s, acc_d):
    c = lax.axis_index("c")
    s = lax.axis_index("s")
    hb = _NB // 2
    rpt = _ROWS_PER_TILE
    pltpu.sync_copy(ones_hbm, ones_v)
    pltpu.sync_copy(zrow_hbm, zbuf)
    pltpu.sync_copy(zbuf, acc_s.at[pl.ds(s * rpt, rpt)])
    pltpu.sync_copy(zbuf, acc_d.at[pl.ds(s * rpt, rpt)])
    plsc.subcore_barrier()
    pltpu.sync_copy(srcp.at[s].at[pl.ds(c * hb, hb)], sidx)
    pltpu.sync_copy(dstp.at[s].at[pl.ds(c * hb, hb)], didx)

    @pl.loop(0, hb)
    def _(b):
        pltpu.sync_copy(ones_v, acc_s.at[sidx.at[b]], add=True)
        pltpu.sync_copy(ones_v, acc_d.at[didx.at[b]], add=True)

    plsc.subcore_barrier()
    pltpu.sync_copy(acc_s.at[pl.ds(s * rpt, rpt)],
                    out_hbm.at[c].at[0].at[pl.ds(s * rpt, rpt)])
    pltpu.sync_copy(acc_d.at[pl.ds(s * rpt, rpt)],
                    out_hbm.at[c].at[1].at[pl.ds(s * rpt, rpt)])


_deg_kernel = functools.partial(
    pl.kernel,
    out_type=jax.ShapeDtypeStruct((2, 2, _NROW, 16), _f32),
    mesh=_MESH,
    compiler_params=_SC_PARAMS,
    scratch_types=[
        pltpu.VMEM((_NB // 2, _K), _i32),
        pltpu.VMEM((_NB // 2, _K), _i32),
        pltpu.VMEM((_K, 16), _f32),
        pltpu.VMEM((_ROWS_PER_TILE, 16), _f32),
        pltpu.VMEM_SHARED((_NROW, 16), _f32),
        pltpu.VMEM_SHARED((_NROW, 16), _f32),
    ],
)(_deg_kernel_body)


def _agg512_body(p_hbm, srcp, dstp, zrow_hbm, out_hbm,
                 sidx, didx, buf0, buf1, acc, sg0, sg1, ss0, ss1,
                 *, do_gather=True, do_scatter=True):
    c = lax.axis_index("c")
    s = lax.axis_index("s")
    rpt = _ROWS_PER_TILE
    pltpu.sync_copy(srcp.at[s], sidx)
    pltpu.sync_copy(dstp.at[s], didx)

    for ci in range(_NCH // 2):
        chunk = c * _NCH // 2 + ci
        tbl = p_hbm.at[chunk]
        och = out_hbm.at[chunk]

        pltpu.sync_copy(zrow_hbm, buf0)
        for j in range(rpt // _K):
            pltpu.sync_copy(buf0, acc.at[pl.ds(s * rpt + j * _K, _K)])
        plsc.subcore_barrier()

        def gstart(b, buf, sem):
            if do_gather:
                pltpu.async_copy(tbl.at[sidx.at[b]], buf, sem)

        def gwait(buf, sem):
            if do_gather:
                pltpu.make_async_copy(tbl.at[sidx.at[0]], buf, sem).wait()

        def sstart(b, buf, sem):
            if do_scatter:
                pltpu.async_copy(buf, acc.at[didx.at[b]], sem, add=True)

        def swait(buf, sem):
            if do_scatter:
                pltpu.make_async_copy(buf, acc.at[didx.at[0]], sem).wait()

        gstart(0, buf0, sg0)

        @pl.loop(0, _NB, step=2)
        def _(b):
            gwait(buf0, sg0)

            @pl.when(b > 0)
            def _():
                swait(buf1, ss1)

            gstart(b + 1, buf1, sg1)
            sstart(b, buf0, ss0)
            gwait(buf1, sg1)
            swait(buf0, ss0)

            @pl.when(b + 2 < _NB)
            def _():
                gstart(b + 2, buf0, sg0)

            sstart(b + 1, buf1, ss1)

        swait(buf1, ss1)
        plsc.subcore_barrier()
        pltpu.sync_copy(acc.at[pl.ds(s * rpt, rpt)], och.at[pl.ds(s * rpt, rpt)])
        plsc.subcore_barrier()


def _gdiag_body(p_hbm, srcp, out_hbm, sidx, buf0, buf1, sg0, sg1, *, ncch):
    c = lax.axis_index("c")
    s = lax.axis_index("s")
    pltpu.sync_copy(srcp.at[s], sidx)
    for ci in range(ncch):
        chunk = c * ncch + ci
        tbl = p_hbm.at[chunk]

        def gstart(b, buf, sem):
            pltpu.async_copy(tbl.at[sidx.at[b]], buf, sem)

        def gwait(buf, sem):
            pltpu.make_async_copy(tbl.at[sidx.at[0]], buf, sem).wait()

        gstart(0, buf0, sg0)

        @pl.loop(0, _NB, step=2)
        def _(b):
            gwait(buf0, sg0)
            gstart(b + 1, buf1, sg1)
            gwait(buf1, sg1)

            @pl.when(b + 2 < _NB)
            def _():
                gstart(b + 2, buf0, sg0)

    pltpu.sync_copy(buf0, out_hbm.at[c * _NSUB + s])


def _make_gdiag(ncch, w, dt, tiled):
    return functools.partial(
        pl.kernel,
        out_type=jax.ShapeDtypeStruct((32, _K, w), dt),
        mesh=_MESH,
        compiler_params=None if tiled else _SC_PARAMS,
        scratch_types=[
            pltpu.VMEM((_NB, _K), _i32),
            pltpu.VMEM((_K, w), dt),
            pltpu.VMEM((_K, w), dt),
            pltpu.SemaphoreType.DMA,
            pltpu.SemaphoreType.DMA,
        ],
    )(functools.partial(_gdiag_body, ncch=ncch))


_gdiag_bf16_64 = _make_gdiag(4, 64, jnp.bfloat16, False)
_gdiag_f32_128 = _make_gdiag(2, 128, _f32, False)
_gdiag_f32_128t = _make_gdiag(2, 128, _f32, True)


def _make_agg512(**kw):
    return functools.partial(
        pl.kernel,
        out_type=jax.ShapeDtypeStruct((_NCH, _NROW, _C), _f32),
        mesh=_MESH,
        compiler_params=_SC_PARAMS,
        scratch_types=[
            pltpu.VMEM((_NB, _K), _i32),
            pltpu.VMEM((_NB, _K), _i32),
            pltpu.VMEM((_K, _C), _f32),
            pltpu.VMEM((_K, _C), _f32),
            pltpu.VMEM_SHARED((_NROW, _C), _f32),
            pltpu.SemaphoreType.DMA,
            pltpu.SemaphoreType.DMA,
            pltpu.SemaphoreType.DMA,
            pltpu.SemaphoreType.DMA,
        ],
    )(functools.partial(_agg512_body, **kw))


_agg512_gonly = _make_agg512(do_scatter=False)
_agg512_sonly = _make_agg512(do_gather=False)

_agg512 = functools.partial(
    pl.kernel,
    out_type=jax.ShapeDtypeStruct((_NCH, _NROW, _C), _f32),
    mesh=_MESH,
    compiler_params=_SC_PARAMS,
    scratch_types=[
        pltpu.VMEM((_NB, _K), _i32),
        pltpu.VMEM((_NB, _K), _i32),
        pltpu.VMEM((_K, _C), _f32),
        pltpu.VMEM((_K, _C), _f32),
        pltpu.VMEM_SHARED((_NROW, _C), _f32),
        pltpu.SemaphoreType.DMA,
        pltpu.SemaphoreType.DMA,
        pltpu.SemaphoreType.DMA,
        pltpu.SemaphoreType.DMA,
    ],
)(_agg512_body)


def _agg16_body(p_hbm, srcp, dstp, zrow_hbm, out_hbm,
                sidx, didx, buf0, buf1, acc, sg0, sg1, ss0, ss1):
    c = lax.axis_index("c")
    s = lax.axis_index("s")
    rpt = _ROWS_PER_TILE
    hb = _NB // 2
    pltpu.sync_copy(srcp.at[s].at[pl.ds(c * hb, hb)], sidx)
    pltpu.sync_copy(dstp.at[s].at[pl.ds(c * hb, hb)], didx)

    pltpu.sync_copy(zrow_hbm, buf0)
    for j in range(rpt // _K):
        pltpu.sync_copy(buf0, acc.at[pl.ds(s * rpt + j * _K, _K)])
    plsc.subcore_barrier()

    def gstart(b, buf, sem):
        pltpu.async_copy(p_hbm.at[sidx.at[b]], buf, sem)

    def gwait(buf, sem):
        pltpu.make_async_copy(p_hbm.at[sidx.at[0]], buf, sem).wait()

    def sstart(b, buf, sem):
        pltpu.async_copy(buf, acc.at[didx.at[b]], sem, add=True)

    def swait(buf, sem):
        pltpu.make_async_copy(buf, acc.at[didx.at[0]], sem).wait()

    gstart(0, buf0, sg0)

    @pl.loop(0, hb, step=2)
    def _(b):
        gwait(buf0, sg0)

        @pl.when(b > 0)
        def _():
            swait(buf1, ss1)

        gstart(b + 1, buf1, sg1)
        sstart(b, buf0, ss0)
        gwait(buf1, sg1)
        swait(buf0, ss0)

        @pl.when(b + 2 < hb)
        def _():
            gstart(b + 2, buf0, sg0)

        sstart(b + 1, buf1, ss1)

    swait(buf1, ss1)
    plsc.subcore_barrier()
    pltpu.sync_copy(acc.at[pl.ds(s * rpt, rpt)],
                    out_hbm.at[c].at[pl.ds(s * rpt, rpt)])


_agg16 = functools.partial(
    pl.kernel,
    out_type=jax.ShapeDtypeStruct((2, _NROW, 16), _f32),
    mesh=_MESH,
    compiler_params=_SC_PARAMS,
    scratch_types=[
        pltpu.VMEM((_NB // 2, _K), _i32),
        pltpu.VMEM((_NB // 2, _K), _i32),
        pltpu.VMEM((_K, 16), _f32),
        pltpu.VMEM((_K, 16), _f32),
        pltpu.VMEM_SHARED((_NROW, 16), _f32),
        pltpu.SemaphoreType.DMA,
        pltpu.SemaphoreType.DMA,
        pltpu.SemaphoreType.DMA,
        pltpu.SemaphoreType.DMA,
    ],
)(_agg16_body)


def _dense1_body(x_ref, w_ref, b_ref, o_ref):
    a = x_ref[0].astype(jnp.bfloat16)
    w = w_ref[0].astype(jnp.bfloat16)
    acc = lax.dot_general(a, w, (((1,), (0,)), ((), ())),
                          preferred_element_type=_f32)
    o_ref[...] = acc + b_ref[0]


def _dense1(xs, ws, bs):
    return pl.pallas_call(
        _dense1_body,
        grid=(2, 5),
        in_specs=[
            pl.BlockSpec((1, 1000, 256), lambda j, i: (j, i, 0)),
            pl.BlockSpec((1, 256, 512), lambda j, i: (j, 0, 0)),
            pl.BlockSpec((1, 1, 512), lambda j, i: (j, 0, 0)),
        ],
        out_specs=pl.BlockSpec((1000, 512), lambda j, i: (j * 5 + i, 0)),
        out_shape=jax.ShapeDtypeStruct((_NROW, 512), _f32),
    )(xs, ws, bs)


def _prep_body(h_ref, dg_ref, p1_ref, nin_ref, nout_ref):
    dg = dg_ref[...]
    d_out = dg[0, 0, :, 0:1] + dg[1, 0, :, 0:1]
    d_in = dg[0, 1, :, 0:1] + dg[1, 1, :, 0:1]
    no = jnp.where(d_out > 0, lax.rsqrt(d_out), 0.0)
    ni = jnp.where(d_in > 0, lax.rsqrt(d_in), 0.0)
    blk = no.shape[0]
    nout_ref[...] = jnp.broadcast_to(no, (blk, _C))
    nin_ref[...] = jnp.broadcast_to(ni, (blk, _C))
    hb = h_ref[...] * no
    for cc in range(_NCH):
        p1_ref[cc] = hb[:, cc * _C:(cc + 1) * _C]


def _prep(h, degp):
    blk = 1024
    return pl.pallas_call(
        _prep_body,
        grid=(_NROW // blk,),
        in_specs=[
            pl.BlockSpec((blk, 512), lambda i: (i, 0)),
            pl.BlockSpec((2, 2, blk, 16), lambda i: (0, 0, i, 0)),
        ],
        out_specs=[
            pl.BlockSpec((_NCH, blk, _C), lambda i: (0, i, 0)),
            pl.BlockSpec((blk, _C), lambda i: (i, 0)),
            pl.BlockSpec((blk, _C), lambda i: (i, 0)),
        ],
        out_shape=[
            jax.ShapeDtypeStruct((_NCH, _NROW, _C), _f32),
            jax.ShapeDtypeStruct((_NROW, _C), _f32),
            jax.ShapeDtypeStruct((_NROW, _C), _f32),
        ],
    )(h, degp)


def _mid_body(a_ref, nin_ref, nout_ref, b_ref, w_ref, o_ref, *, nwo):
    nin = nin_ref[...]
    nout = nout_ref[...]
    blk = nin.shape[0]
    acc = jnp.zeros((blk, nwo), _f32)
    for cc in range(_NCH):
        hc = jnp.maximum(a_ref[cc] * nin + b_ref[cc], 0.0) * nout
        acc = acc + lax.dot_general(
            hc.astype(jnp.bfloat16), w_ref[cc].astype(jnp.bfloat16),
            (((1,), (0,)), ((), ())), preferred_element_type=_f32)
    if nwo == 512:
        for cc in range(_NCH):
            o_ref[cc] = acc[:, cc * _C:(cc + 1) * _C]
    else:
        o_ref[...] = acc


def _mid(a, nin, nout, b, w):
    blk = 1024
    nwo = w.shape[2]
    if nwo == 512:
        out_spec = pl.BlockSpec((_NCH, blk, _C), lambda i: (0, i, 0))
        out_shape = jax.ShapeDtypeStruct((_NCH, _NROW, _C), _f32)
    else:
        out_spec = pl.BlockSpec((blk, nwo), lambda i: (i, 0))
        out_shape = jax.ShapeDtypeStruct((_NROW, nwo), _f32)
    return pl.pallas_call(
        functools.partial(_mid_body, nwo=nwo),
        grid=(_NROW // blk,),
        in_specs=[
            pl.BlockSpec((_NCH, blk, _C), lambda i: (0, i, 0)),
            pl.BlockSpec((blk, _C), lambda i: (i, 0)),
            pl.BlockSpec((blk, _C), lambda i: (i, 0)),
            pl.BlockSpec((_NCH, 1, _C), lambda i: (0, 0, 0)),
            pl.BlockSpec((_NCH, _C, nwo), lambda i: (0, 0, 0)),
        ],
        out_specs=out_spec,
        out_shape=out_shape,
    )(a, nin, nout, b, w)


def _final_body(a_ref, nin_ref, b_ref, o_ref):
    a = a_ref[0] + a_ref[1]
    o_ref[...] = a * nin_ref[:, :16] + b_ref[...]


def _final(a3p, nin, b2):
    blk = 1000
    return pl.pallas_call(
        _final_body,
        grid=(_N // blk,),
        in_specs=[
            pl.BlockSpec((2, blk, 16), lambda i: (0, i, 0)),
            pl.BlockSpec((blk, _C), lambda i: (i, 0)),
            pl.BlockSpec((1, 16), lambda i: (0, 0)),
        ],
        out_specs=pl.BlockSpec((blk, 16), lambda i: (i, 0)),
        out_shape=jax.ShapeDtypeStruct((_N, 16), _f32),
    )(a3p, nin, b2)


def kernel(x0, x1, edge_index, fc0_w, fc0_b, fc1_w, fc1_b, conv0_b, conv1_w,
           conv1_b, conv2_w, conv2_b):
    src = edge_index[0]
    dst = edge_index[1]
    pad = jnp.full((_EPAD - _E,), _N, _i32)
    srcp = jnp.concatenate([src, pad]).reshape(_NSUB, _NB, _K)
    dstp = jnp.concatenate([dst, pad]).reshape(_NSUB, _NB, _K)

    ones128 = jnp.ones((_K, 16), _f32)
    zrow1 = jnp.zeros((_ROWS_PER_TILE, 16), _f32)
    zrow128 = jnp.zeros((_K, _C), _f32)
    zrow16 = jnp.zeros((_K, 16), _f32)

    xs = jnp.stack([x0, x1])
    ws = jnp.stack([fc0_w, fc1_w])
    bs = jnp.stack([fc0_b, fc1_b]).reshape(2, 1, 512)

    degp = _deg_kernel(srcp, dstp, ones128, zrow1)
    h = _dense1(xs, ws, bs)
    p1, nin, nout = _prep(h, degp)

    a1 = _agg512(p1, srcp, dstp, zrow128)
    perm = jnp.argsort(src)
    ssrc = jnp.concatenate([src[perm], pad]).reshape(_NSUB, _NB, _K)
    sdst = jnp.concatenate([dst[perm], pad]).reshape(_NSUB, _NB, _K)
    d1 = _agg512(p1, ssrc, sdst, zrow128)
    a1 = a1 + 0.0 * d1
    p2 = _mid(a1, nin, nout, conv0_b.reshape(_NCH, 1, _C),
              conv1_w.reshape(_NCH, _C, 512))
    a2 = _agg512(p2, srcp, dstp, zrow128)
    p3 = _mid(a2, nin, nout, conv1_b.reshape(_NCH, 1, _C),
              conv2_w.reshape(_NCH, _C, 16))
    a3p = _agg16(p3, srcp, dstp, zrow16)
    out = _final(a3p, nin, conv2_b.reshape(1, 16))
    return out


# speedup vs baseline: 2.3641x; 1.8747x over previous
import functools

import jax
import jax.numpy as jnp
from jax import lax
from jax.experimental import pallas as pl
from jax.experimental.pallas import tpu as pltpu
from jax.experimental.pallas import tpu_sc as plsc

_N = 10000
_NROW = 10240
_E = 160000
_NSUB = 16
_K = 128
_NB = 80
_EPAD = _NSUB * _NB * _K
_C = 64
_NCH = 512 // _C
_ROWS_PER_TILE = _NROW // _NSUB
_NBUF = 4

_f32 = jnp.float32
_i32 = jnp.int32

_MESH = plsc.VectorSubcoreMesh(core_axis_name="c", subcore_axis_name="s")
_SC_PARAMS = pltpu.CompilerParams(use_tc_tiling_on_sc=False)


def _deg_kernel_body(srcp, dstp, ones_hbm, zrow_hbm, out_hbm,
                     sidx, didx, ones_v, zbuf, acc_s, acc_d):
    c = lax.axis_index("c")
    s = lax.axis_index("s")
    hb = _NB // 2
    rpt = _ROWS_PER_TILE
    pltpu.sync_copy(ones_hbm, ones_v)
    pltpu.sync_copy(zrow_hbm, zbuf)
    pltpu.sync_copy(zbuf, acc_s.at[pl.ds(s * rpt, rpt)])
    pltpu.sync_copy(zbuf, acc_d.at[pl.ds(s * rpt, rpt)])
    plsc.subcore_barrier()
    pltpu.sync_copy(srcp.at[s].at[pl.ds(c * hb, hb)], sidx)
    pltpu.sync_copy(dstp.at[s].at[pl.ds(c * hb, hb)], didx)

    @pl.loop(0, hb)
    def _(b):
        pltpu.sync_copy(ones_v, acc_s.at[sidx.at[b]], add=True)
        pltpu.sync_copy(ones_v, acc_d.at[didx.at[b]], add=True)

    plsc.subcore_barrier()
    pltpu.sync_copy(acc_s.at[pl.ds(s * rpt, rpt)],
                    out_hbm.at[c].at[0].at[pl.ds(s * rpt, rpt)])
    pltpu.sync_copy(acc_d.at[pl.ds(s * rpt, rpt)],
                    out_hbm.at[c].at[1].at[pl.ds(s * rpt, rpt)])


_deg_kernel = functools.partial(
    pl.kernel,
    out_type=jax.ShapeDtypeStruct((2, 2, _NROW, 16), _f32),
    mesh=_MESH,
    compiler_params=_SC_PARAMS,
    scratch_types=[
        pltpu.VMEM((_NB // 2, _K), _i32),
        pltpu.VMEM((_NB // 2, _K), _i32),
        pltpu.VMEM((_K, 16), _f32),
        pltpu.VMEM((_ROWS_PER_TILE, 16), _f32),
        pltpu.VMEM_SHARED((_NROW, 16), _f32),
        pltpu.VMEM_SHARED((_NROW, 16), _f32),
    ],
)(_deg_kernel_body)


def _agg512_body(p_hbm, srcp, dstp, zrow_hbm, out_hbm,
                 sidx, didx, b0, b1, b2, b3, acc,
                 g0, g1, g2, g3, s0, s1, s2, s3):
    bufs = [b0, b1, b2, b3]
    gsems = [g0, g1, g2, g3]
    ssems = [s0, s1, s2, s3]
    c = lax.axis_index("c")
    s = lax.axis_index("s")
    rpt = _ROWS_PER_TILE
    pltpu.sync_copy(srcp.at[s], sidx)
    pltpu.sync_copy(dstp.at[s], didx)

    for ci in range(_NCH // 2):
        chunk = c * _NCH // 2 + ci
        tbl = p_hbm.at[chunk]
        och = out_hbm.at[chunk]

        pltpu.sync_copy(zrow_hbm, bufs[0])
        for j in range(rpt // _K):
            pltpu.sync_copy(bufs[0], acc.at[pl.ds(s * rpt + j * _K, _K)])
        plsc.subcore_barrier()

        def gstart(b, buf, sem):
            pltpu.async_copy(tbl.at[sidx.at[b]], buf, sem)

        def gwait(buf, sem):
            pltpu.make_async_copy(tbl.at[sidx.at[0]], buf, sem).wait()

        def sstart(b, buf, sem):
            pltpu.async_copy(buf, acc.at[didx.at[b]], sem, add=True)

        def swait(buf, sem):
            pltpu.make_async_copy(buf, acc.at[didx.at[0]], sem).wait()

        for k in range(_NBUF):
            gstart(k, bufs[k], gsems[k])

        @pl.loop(0, _NB, step=_NBUF)
        def _(b):
            for k in range(_NBUF):
                gwait(bufs[k], gsems[k])
                sstart(b + k, bufs[k], ssems[k])
            for k in range(_NBUF):
                swait(bufs[k], ssems[k])

                @pl.when(b + _NBUF + k < _NB)
                def _():
                    gstart(b + _NBUF + k, bufs[k], gsems[k])

        plsc.subcore_barrier()
        pltpu.sync_copy(acc.at[pl.ds(s * rpt, rpt)], och.at[pl.ds(s * rpt, rpt)])
        plsc.subcore_barrier()


_agg512 = functools.partial(
    pl.kernel,
    out_type=jax.ShapeDtypeStruct((_NCH, _NROW, _C), _f32),
    mesh=_MESH,
    compiler_params=_SC_PARAMS,
    scratch_types=(
        [pltpu.VMEM((_NB, _K), _i32)] * 2
        + [pltpu.VMEM((_K, _C), _f32)] * _NBUF
        + [pltpu.VMEM_SHARED((_NROW, _C), _f32)]
        + [pltpu.SemaphoreType.DMA] * (2 * _NBUF)
    ),
)(_agg512_body)


def _agg16_body(p_hbm, srcp, dstp, zrow_hbm, out_hbm,
                sidx, didx, buf0, buf1, acc, sg0, sg1, ss0, ss1):
    c = lax.axis_index("c")
    s = lax.axis_index("s")
    rpt = _ROWS_PER_TILE
    hb = _NB // 2
    pltpu.sync_copy(srcp.at[s].at[pl.ds(c * hb, hb)], sidx)
    pltpu.sync_copy(dstp.at[s].at[pl.ds(c * hb, hb)], didx)

    pltpu.sync_copy(zrow_hbm, buf0)
    for j in range(rpt // _K):
        pltpu.sync_copy(buf0, acc.at[pl.ds(s * rpt + j * _K, _K)])
    plsc.subcore_barrier()

    def gstart(b, buf, sem):
        pltpu.async_copy(p_hbm.at[sidx.at[b]], buf, sem)

    def gwait(buf, sem):
        pltpu.make_async_copy(p_hbm.at[sidx.at[0]], buf, sem).wait()

    def sstart(b, buf, sem):
        pltpu.async_copy(buf, acc.at[didx.at[b]], sem, add=True)

    def swait(buf, sem):
        pltpu.make_async_copy(buf, acc.at[didx.at[0]], sem).wait()

    gstart(0, buf0, sg0)

    @pl.loop(0, hb, step=2)
    def _(b):
        gwait(buf0, sg0)

        @pl.when(b > 0)
        def _():
            swait(buf1, ss1)

        gstart(b + 1, buf1, sg1)
        sstart(b, buf0, ss0)
        gwait(buf1, sg1)
        swait(buf0, ss0)

        @pl.when(b + 2 < hb)
        def _():
            gstart(b + 2, buf0, sg0)

        sstart(b + 1, buf1, ss1)

    swait(buf1, ss1)
    plsc.subcore_barrier()
    pltpu.sync_copy(acc.at[pl.ds(s * rpt, rpt)],
                    out_hbm.at[c].at[pl.ds(s * rpt, rpt)])


_agg16 = functools.partial(
    pl.kernel,
    out_type=jax.ShapeDtypeStruct((2, _NROW, 16), _f32),
    mesh=_MESH,
    compiler_params=_SC_PARAMS,
    scratch_types=[
        pltpu.VMEM((_NB // 2, _K), _i32),
        pltpu.VMEM((_NB // 2, _K), _i32),
        pltpu.VMEM((_K, 16), _f32),
        pltpu.VMEM((_K, 16), _f32),
        pltpu.VMEM_SHARED((_NROW, 16), _f32),
        pltpu.SemaphoreType.DMA,
        pltpu.SemaphoreType.DMA,
        pltpu.SemaphoreType.DMA,
        pltpu.SemaphoreType.DMA,
    ],
)(_agg16_body)


def _dense1_body(x_ref, w_ref, b_ref, o_ref):
    a = x_ref[0].astype(jnp.bfloat16)
    w = w_ref[0].astype(jnp.bfloat16)
    acc = lax.dot_general(a, w, (((1,), (0,)), ((), ())),
                          preferred_element_type=_f32)
    o_ref[...] = acc + b_ref[0]


def _dense1(xs, ws, bs):
    return pl.pallas_call(
        _dense1_body,
        grid=(2, 5),
        in_specs=[
            pl.BlockSpec((1, 1000, 256), lambda j, i: (j, i, 0)),
            pl.BlockSpec((1, 256, 512), lambda j, i: (j, 0, 0)),
            pl.BlockSpec((1, 1, 512), lambda j, i: (j, 0, 0)),
        ],
        out_specs=pl.BlockSpec((1000, 512), lambda j, i: (j * 5 + i, 0)),
        out_shape=jax.ShapeDtypeStruct((_NROW, 512), _f32),
    )(xs, ws, bs)


def _prep_body(h_ref, dg_ref, p1_ref, nin_ref, nout_ref):
    dg = dg_ref[...]
    d_out = dg[0, 0, :, 0:1] + dg[1, 0, :, 0:1]
    d_in = dg[0, 1, :, 0:1] + dg[1, 1, :, 0:1]
    no = jnp.where(d_out > 0, lax.rsqrt(d_out), 0.0)
    ni = jnp.where(d_in > 0, lax.rsqrt(d_in), 0.0)
    blk = no.shape[0]
    nout_ref[...] = jnp.broadcast_to(no, (blk, _C))
    nin_ref[...] = jnp.broadcast_to(ni, (blk, _C))
    hb = h_ref[...] * no
    for cc in range(_NCH):
        p1_ref[cc] = hb[:, cc * _C:(cc + 1) * _C]


def _prep(h, degp):
    blk = 1024
    return pl.pallas_call(
        _prep_body,
        grid=(_NROW // blk,),
        in_specs=[
            pl.BlockSpec((blk, 512), lambda i: (i, 0)),
            pl.BlockSpec((2, 2, blk, 16), lambda i: (0, 0, i, 0)),
        ],
        out_specs=[
            pl.BlockSpec((_NCH, blk, _C), lambda i: (0, i, 0)),
            pl.BlockSpec((blk, _C), lambda i: (i, 0)),
            pl.BlockSpec((blk, _C), lambda i: (i, 0)),
        ],
        out_shape=[
            jax.ShapeDtypeStruct((_NCH, _NROW, _C), _f32),
            jax.ShapeDtypeStruct((_NROW, _C), _f32),
            jax.ShapeDtypeStruct((_NROW, _C), _f32),
        ],
    )(h, degp)


def _mid_body(a_ref, nin_ref, nout_ref, b_ref, w_ref, o_ref, *, nwo):
    nin = nin_ref[...]
    nout = nout_ref[...]
    blk = nin.shape[0]
    acc = jnp.zeros((blk, nwo), _f32)
    for cc in range(_NCH):
        hc = jnp.maximum(a_ref[cc] * nin + b_ref[cc], 0.0) * nout
        acc = acc + lax.dot_general(
            hc.astype(jnp.bfloat16), w_ref[cc].astype(jnp.bfloat16),
            (((1,), (0,)), ((), ())), preferred_element_type=_f32)
    if nwo == 512:
        for cc in range(_NCH):
            o_ref[cc] = acc[:, cc * _C:(cc + 1) * _C]
    else:
        o_ref[...] = acc


def _mid(a, nin, nout, b, w):
    blk = 1024
    nwo = w.shape[2]
    if nwo == 512:
        out_spec = pl.BlockSpec((_NCH, blk, _C), lambda i: (0, i, 0))
        out_shape = jax.ShapeDtypeStruct((_NCH, _NROW, _C), _f32)
    else:
        out_spec = pl.BlockSpec((blk, nwo), lambda i: (i, 0))
        out_shape = jax.ShapeDtypeStruct((_NROW, nwo), _f32)
    return pl.pallas_call(
        functools.partial(_mid_body, nwo=nwo),
        grid=(_NROW // blk,),
        in_specs=[
            pl.BlockSpec((_NCH, blk, _C), lambda i: (0, i, 0)),
            pl.BlockSpec((blk, _C), lambda i: (i, 0)),
            pl.BlockSpec((blk, _C), lambda i: (i, 0)),
            pl.BlockSpec((_NCH, 1, _C), lambda i: (0, 0, 0)),
            pl.BlockSpec((_NCH, _C, nwo), lambda i: (0, 0, 0)),
        ],
        out_specs=out_spec,
        out_shape=out_shape,
    )(a, nin, nout, b, w)


def _final_body(a_ref, nin_ref, b_ref, o_ref):
    a = a_ref[0] + a_ref[1]
    o_ref[...] = a * nin_ref[:, :16] + b_ref[...]


def _final(a3p, nin, b2):
    blk = 1000
    return pl.pallas_call(
        _final_body,
        grid=(_N // blk,),
        in_specs=[
            pl.BlockSpec((2, blk, 16), lambda i: (0, i, 0)),
            pl.BlockSpec((blk, _C), lambda i: (i, 0)),
            pl.BlockSpec((1, 16), lambda i: (0, 0)),
        ],
        out_specs=pl.BlockSpec((blk, 16), lambda i: (i, 0)),
        out_shape=jax.ShapeDtypeStruct((_N, 16), _f32),
    )(a3p, nin, b2)


def kernel(x0, x1, edge_index, fc0_w, fc0_b, fc1_w, fc1_b, conv0_b, conv1_w,
           conv1_b, conv2_w, conv2_b):
    src = edge_index[0]
    dst = edge_index[1]
    pad = jnp.full((_EPAD - _E,), _N, _i32)
    srcp = jnp.concatenate([src, pad]).reshape(_NSUB, _NB, _K)
    dstp = jnp.concatenate([dst, pad]).reshape(_NSUB, _NB, _K)

    ones128 = jnp.ones((_K, 16), _f32)
    zrow1 = jnp.zeros((_ROWS_PER_TILE, 16), _f32)
    zrow128 = jnp.zeros((_K, _C), _f32)
    zrow16 = jnp.zeros((_K, 16), _f32)

    xs = jnp.stack([x0, x1])
    ws = jnp.stack([fc0_w, fc1_w])
    bs = jnp.stack([fc0_b, fc1_b]).reshape(2, 1, 512)

    degp = _deg_kernel(srcp, dstp, ones128, zrow1)
    h = _dense1(xs, ws, bs)
    p1, nin, nout = _prep(h, degp)

    a1 = _agg512(p1, srcp, dstp, zrow128)
    p2 = _mid(a1, nin, nout, conv0_b.reshape(_NCH, 1, _C),
              conv1_w.reshape(_NCH, _C, 512))
    a2 = _agg512(p2, srcp, dstp, zrow128)
    p3 = _mid(a2, nin, nout, conv1_b.reshape(_NCH, 1, _C),
              conv2_w.reshape(_NCH, _C, 16))
    a3p = _agg16(p3, srcp, dstp, zrow16)
    out = _final(a3p, nin, conv2_b.reshape(1, 16))
    return out


# speedup vs baseline: 2.4266x; 1.0264x over previous
import functools

import jax
import jax.numpy as jnp
from jax import lax
from jax.experimental import pallas as pl
from jax.experimental.pallas import tpu as pltpu
from jax.experimental.pallas import tpu_sc as plsc

_N = 10000
_NROW = 10240
_E = 160000
_NSUB = 16
_K = 128
_NB = 80
_EPAD = _NSUB * _NB * _K
_C = 64
_NCH = 512 // _C
_ROWS_PER_TILE = _NROW // _NSUB
_NBUF = 8

_f32 = jnp.float32
_i32 = jnp.int32

_MESH = plsc.VectorSubcoreMesh(core_axis_name="c", subcore_axis_name="s")
_SC_PARAMS = pltpu.CompilerParams(use_tc_tiling_on_sc=False)


def _deg_kernel_body(srcp, dstp, ones_hbm, zrow_hbm, out_hbm,
                     sidx, didx, ones_v, zbuf, acc_s, acc_d):
    c = lax.axis_index("c")
    s = lax.axis_index("s")
    hb = _NB // 2
    rpt = _ROWS_PER_TILE
    pltpu.sync_copy(ones_hbm, ones_v)
    pltpu.sync_copy(zrow_hbm, zbuf)
    pltpu.sync_copy(zbuf, acc_s.at[pl.ds(s * rpt, rpt)])
    pltpu.sync_copy(zbuf, acc_d.at[pl.ds(s * rpt, rpt)])
    plsc.subcore_barrier()
    pltpu.sync_copy(srcp.at[s].at[pl.ds(c * hb, hb)], sidx)
    pltpu.sync_copy(dstp.at[s].at[pl.ds(c * hb, hb)], didx)

    @pl.loop(0, hb)
    def _(b):
        pltpu.sync_copy(ones_v, acc_s.at[sidx.at[b]], add=True)
        pltpu.sync_copy(ones_v, acc_d.at[didx.at[b]], add=True)

    plsc.subcore_barrier()
    pltpu.sync_copy(acc_s.at[pl.ds(s * rpt, rpt)],
                    out_hbm.at[c].at[0].at[pl.ds(s * rpt, rpt)])
    pltpu.sync_copy(acc_d.at[pl.ds(s * rpt, rpt)],
                    out_hbm.at[c].at[1].at[pl.ds(s * rpt, rpt)])


_deg_kernel = functools.partial(
    pl.kernel,
    out_type=jax.ShapeDtypeStruct((2, 2, _NROW, 16), _f32),
    mesh=_MESH,
    compiler_params=_SC_PARAMS,
    scratch_types=[
        pltpu.VMEM((_NB // 2, _K), _i32),
        pltpu.VMEM((_NB // 2, _K), _i32),
        pltpu.VMEM((_K, 16), _f32),
        pltpu.VMEM((_ROWS_PER_TILE, 16), _f32),
        pltpu.VMEM_SHARED((_NROW, 16), _f32),
        pltpu.VMEM_SHARED((_NROW, 16), _f32),
    ],
)(_deg_kernel_body)


def _agg512_body(p_hbm, srcp, dstp, zrow_hbm, out_hbm,
                 sidx, didx, b0, b1, b2, b3, b4, b5, b6, b7, acc,
                 g0, g1, g2, g3, g4, g5, g6, g7,
                 s0, s1, s2, s3, s4, s5, s6, s7):
    bufs = [b0, b1, b2, b3, b4, b5, b6, b7]
    gsems = [g0, g1, g2, g3, g4, g5, g6, g7]
    ssems = [s0, s1, s2, s3, s4, s5, s6, s7]
    c = lax.axis_index("c")
    s = lax.axis_index("s")
    rpt = _ROWS_PER_TILE
    pltpu.sync_copy(srcp.at[s], sidx)
    pltpu.sync_copy(dstp.at[s], didx)

    for ci in range(_NCH // 2):
        chunk = c * _NCH // 2 + ci
        tbl = p_hbm.at[chunk]
        och = out_hbm.at[chunk]

        pltpu.sync_copy(zrow_hbm, bufs[0])
        for j in range(rpt // _K):
            pltpu.sync_copy(bufs[0], acc.at[pl.ds(s * rpt + j * _K, _K)])
        plsc.subcore_barrier()

        def gstart(b, buf, sem):
            pltpu.async_copy(tbl.at[sidx.at[b]], buf, sem)

        def gwait(buf, sem):
            pltpu.make_async_copy(tbl.at[sidx.at[0]], buf, sem).wait()

        def sstart(b, buf, sem):
            pltpu.async_copy(buf, acc.at[didx.at[b]], sem, add=True)

        def swait(buf, sem):
            pltpu.make_async_copy(buf, acc.at[didx.at[0]], sem).wait()

        for k in range(_NBUF):
            gstart(k, bufs[k], gsems[k])

        @pl.loop(0, _NB, step=_NBUF)
        def _(b):
            for k in range(_NBUF):
                gwait(bufs[k], gsems[k])
                sstart(b + k, bufs[k], ssems[k])
            for k in range(_NBUF):
                swait(bufs[k], ssems[k])

                @pl.when(b + _NBUF + k < _NB)
                def _():
                    gstart(b + _NBUF + k, bufs[k], gsems[k])

        plsc.subcore_barrier()
        pltpu.sync_copy(acc.at[pl.ds(s * rpt, rpt)], och.at[pl.ds(s * rpt, rpt)])
        plsc.subcore_barrier()


_agg512 = functools.partial(
    pl.kernel,
    out_type=jax.ShapeDtypeStruct((_NCH, _NROW, _C), _f32),
    mesh=_MESH,
    compiler_params=_SC_PARAMS,
    scratch_types=(
        [pltpu.VMEM((_NB, _K), _i32)] * 2
        + [pltpu.VMEM((_K, _C), _f32)] * _NBUF
        + [pltpu.VMEM_SHARED((_NROW, _C), _f32)]
        + [pltpu.SemaphoreType.DMA] * (2 * _NBUF)
    ),
)(_agg512_body)


def _agg16_body(p_hbm, srcp, dstp, zrow_hbm, out_hbm,
                sidx, didx, buf0, buf1, acc, sg0, sg1, ss0, ss1):
    c = lax.axis_index("c")
    s = lax.axis_index("s")
    rpt = _ROWS_PER_TILE
    hb = _NB // 2
    pltpu.sync_copy(srcp.at[s].at[pl.ds(c * hb, hb)], sidx)
    pltpu.sync_copy(dstp.at[s].at[pl.ds(c * hb, hb)], didx)

    pltpu.sync_copy(zrow_hbm, buf0)
    for j in range(rpt // _K):
        pltpu.sync_copy(buf0, acc.at[pl.ds(s * rpt + j * _K, _K)])
    plsc.subcore_barrier()

    def gstart(b, buf, sem):
        pltpu.async_copy(p_hbm.at[sidx.at[b]], buf, sem)

    def gwait(buf, sem):
        pltpu.make_async_copy(p_hbm.at[sidx.at[0]], buf, sem).wait()

    def sstart(b, buf, sem):
        pltpu.async_copy(buf, acc.at[didx.at[b]], sem, add=True)

    def swait(buf, sem):
        pltpu.make_async_copy(buf, acc.at[didx.at[0]], sem).wait()

    gstart(0, buf0, sg0)

    @pl.loop(0, hb, step=2)
    def _(b):
        gwait(buf0, sg0)

        @pl.when(b > 0)
        def _():
            swait(buf1, ss1)

        gstart(b + 1, buf1, sg1)
        sstart(b, buf0, ss0)
        gwait(buf1, sg1)
        swait(buf0, ss0)

        @pl.when(b + 2 < hb)
        def _():
            gstart(b + 2, buf0, sg0)

        sstart(b + 1, buf1, ss1)

    swait(buf1, ss1)
    plsc.subcore_barrier()
    pltpu.sync_copy(acc.at[pl.ds(s * rpt, rpt)],
                    out_hbm.at[c].at[pl.ds(s * rpt, rpt)])


_agg16 = functools.partial(
    pl.kernel,
    out_type=jax.ShapeDtypeStruct((2, _NROW, 16), _f32),
    mesh=_MESH,
    compiler_params=_SC_PARAMS,
    scratch_types=[
        pltpu.VMEM((_NB // 2, _K), _i32),
        pltpu.VMEM((_NB // 2, _K), _i32),
        pltpu.VMEM((_K, 16), _f32),
        pltpu.VMEM((_K, 16), _f32),
        pltpu.VMEM_SHARED((_NROW, 16), _f32),
        pltpu.SemaphoreType.DMA,
        pltpu.SemaphoreType.DMA,
        pltpu.SemaphoreType.DMA,
        pltpu.SemaphoreType.DMA,
    ],
)(_agg16_body)


def _dense1_body(x_ref, w_ref, b_ref, o_ref):
    a = x_ref[0].astype(jnp.bfloat16)
    w = w_ref[0].astype(jnp.bfloat16)
    acc = lax.dot_general(a, w, (((1,), (0,)), ((), ())),
                          preferred_element_type=_f32)
    o_ref[...] = acc + b_ref[0]


def _dense1(xs, ws, bs):
    return pl.pallas_call(
        _dense1_body,
        grid=(2, 5),
        in_specs=[
            pl.BlockSpec((1, 1000, 256), lambda j, i: (j, i, 0)),
            pl.BlockSpec((1, 256, 512), lambda j, i: (j, 0, 0)),
            pl.BlockSpec((1, 1, 512), lambda j, i: (j, 0, 0)),
        ],
        out_specs=pl.BlockSpec((1000, 512), lambda j, i: (j * 5 + i, 0)),
        out_shape=jax.ShapeDtypeStruct((_NROW, 512), _f32),
    )(xs, ws, bs)


def _prep_body(h_ref, dg_ref, p1_ref, nin_ref, nout_ref):
    dg = dg_ref[...]
    d_out = dg[0, 0, :, 0:1] + dg[1, 0, :, 0:1]
    d_in = dg[0, 1, :, 0:1] + dg[1, 1, :, 0:1]
    no = jnp.where(d_out > 0, lax.rsqrt(d_out), 0.0)
    ni = jnp.where(d_in > 0, lax.rsqrt(d_in), 0.0)
    blk = no.shape[0]
    nout_ref[...] = jnp.broadcast_to(no, (blk, _C))
    nin_ref[...] = jnp.broadcast_to(ni, (blk, _C))
    hb = h_ref[...] * no
    for cc in range(_NCH):
        p1_ref[cc] = hb[:, cc * _C:(cc + 1) * _C]


def _prep(h, degp):
    blk = 1024
    return pl.pallas_call(
        _prep_body,
        grid=(_NROW // blk,),
        in_specs=[
            pl.BlockSpec((blk, 512), lambda i: (i, 0)),
            pl.BlockSpec((2, 2, blk, 16), lambda i: (0, 0, i, 0)),
        ],
        out_specs=[
            pl.BlockSpec((_NCH, blk, _C), lambda i: (0, i, 0)),
            pl.BlockSpec((blk, _C), lambda i: (i, 0)),
            pl.BlockSpec((blk, _C), lambda i: (i, 0)),
        ],
        out_shape=[
            jax.ShapeDtypeStruct((_NCH, _NROW, _C), _f32),
            jax.ShapeDtypeStruct((_NROW, _C), _f32),
            jax.ShapeDtypeStruct((_NROW, _C), _f32),
        ],
    )(h, degp)


def _mid_body(a_ref, nin_ref, nout_ref, b_ref, w_ref, o_ref, *, nwo):
    nin = nin_ref[...]
    nout = nout_ref[...]
    blk = nin.shape[0]
    acc = jnp.zeros((blk, nwo), _f32)
    for cc in range(_NCH):
        hc = jnp.maximum(a_ref[cc] * nin + b_ref[cc], 0.0) * nout
        acc = acc + lax.dot_general(
            hc.astype(jnp.bfloat16), w_ref[cc].astype(jnp.bfloat16),
            (((1,), (0,)), ((), ())), preferred_element_type=_f32)
    if nwo == 512:
        for cc in range(_NCH):
            o_ref[cc] = acc[:, cc * _C:(cc + 1) * _C]
    else:
        o_ref[...] = acc


def _mid(a, nin, nout, b, w):
    blk = 1024
    nwo = w.shape[2]
    if nwo == 512:
        out_spec = pl.BlockSpec((_NCH, blk, _C), lambda i: (0, i, 0))
        out_shape = jax.ShapeDtypeStruct((_NCH, _NROW, _C), _f32)
    else:
        out_spec = pl.BlockSpec((blk, nwo), lambda i: (i, 0))
        out_shape = jax.ShapeDtypeStruct((_NROW, nwo), _f32)
    return pl.pallas_call(
        functools.partial(_mid_body, nwo=nwo),
        grid=(_NROW // blk,),
        in_specs=[
            pl.BlockSpec((_NCH, blk, _C), lambda i: (0, i, 0)),
            pl.BlockSpec((blk, _C), lambda i: (i, 0)),
            pl.BlockSpec((blk, _C), lambda i: (i, 0)),
            pl.BlockSpec((_NCH, 1, _C), lambda i: (0, 0, 0)),
            pl.BlockSpec((_NCH, _C, nwo), lambda i: (0, 0, 0)),
        ],
        out_specs=out_spec,
        out_shape=out_shape,
    )(a, nin, nout, b, w)


def _final_body(a_ref, nin_ref, b_ref, o_ref):
    a = a_ref[0] + a_ref[1]
    o_ref[...] = a * nin_ref[:, :16] + b_ref[...]


def _final(a3p, nin, b2):
    blk = 1000
    return pl.pallas_call(
        _final_body,
        grid=(_N // blk,),
        in_specs=[
            pl.BlockSpec((2, blk, 16), lambda i: (0, i, 0)),
            pl.BlockSpec((blk, _C), lambda i: (i, 0)),
            pl.BlockSpec((1, 16), lambda i: (0, 0)),
        ],
        out_specs=pl.BlockSpec((blk, 16), lambda i: (i, 0)),
        out_shape=jax.ShapeDtypeStruct((_N, 16), _f32),
    )(a3p, nin, b2)


def kernel(x0, x1, edge_index, fc0_w, fc0_b, fc1_w, fc1_b, conv0_b, conv1_w,
           conv1_b, conv2_w, conv2_b):
    src = edge_index[0]
    dst = edge_index[1]
    pad = jnp.full((_EPAD - _E,), _N, _i32)
    srcp = jnp.concatenate([src, pad]).reshape(_NSUB, _NB, _K)
    dstp = jnp.concatenate([dst, pad]).reshape(_NSUB, _NB, _K)

    ones128 = jnp.ones((_K, 16), _f32)
    zrow1 = jnp.zeros((_ROWS_PER_TILE, 16), _f32)
    zrow128 = jnp.zeros((_K, _C), _f32)
    zrow16 = jnp.zeros((_K, 16), _f32)

    xs = jnp.stack([x0, x1])
    ws = jnp.stack([fc0_w, fc1_w])
    bs = jnp.stack([fc0_b, fc1_b]).reshape(2, 1, 512)

    degp = _deg_kernel(srcp, dstp, ones128, zrow1)
    h = _dense1(xs, ws, bs)
    p1, nin, nout = _prep(h, degp)

    a1 = _agg512(p1, srcp, dstp, zrow128)
    p2 = _mid(a1, nin, nout, conv0_b.reshape(_NCH, 1, _C),
              conv1_w.reshape(_NCH, _C, 512))
    a2 = _agg512(p2, srcp, dstp, zrow128)
    p3 = _mid(a2, nin, nout, conv1_b.reshape(_NCH, 1, _C),
              conv2_w.reshape(_NCH, _C, 16))
    a3p = _agg16(p3, srcp, dstp, zrow16)
    out = _final(a3p, nin, conv2_b.reshape(1, 16))
    return out


# speedup vs baseline: 2.4969x; 1.0290x over previous
import functools

import jax
import jax.numpy as jnp
from jax import lax
from jax.experimental import pallas as pl
from jax.experimental.pallas import tpu as pltpu
from jax.experimental.pallas import tpu_sc as plsc

_N = 10000
_NROW = 10240
_E = 160000
_NSUB = 16
_K = 128
_NB = 80
_EPAD = _NSUB * _NB * _K
_C = 64
_NCH = 512 // _C
_ROWS_PER_TILE = _NROW // _NSUB
_NBUF = 8

_f32 = jnp.float32
_i32 = jnp.int32

_MESH = plsc.VectorSubcoreMesh(core_axis_name="c", subcore_axis_name="s")
_SC_PARAMS = pltpu.CompilerParams(use_tc_tiling_on_sc=False)


def _deg_kernel_body(srcp, dstp, ones_hbm, zrow_hbm, out_hbm,
                     sidx, didx, ones_v, zbuf, acc_s, acc_d):
    c = lax.axis_index("c")
    s = lax.axis_index("s")
    hb = _NB // 2
    rpt = _ROWS_PER_TILE
    pltpu.sync_copy(ones_hbm, ones_v)
    pltpu.sync_copy(zrow_hbm, zbuf)
    pltpu.sync_copy(zbuf, acc_s.at[pl.ds(s * rpt, rpt)])
    pltpu.sync_copy(zbuf, acc_d.at[pl.ds(s * rpt, rpt)])
    plsc.subcore_barrier()
    pltpu.sync_copy(srcp.at[s].at[pl.ds(c * hb, hb)], sidx)
    pltpu.sync_copy(dstp.at[s].at[pl.ds(c * hb, hb)], didx)

    @pl.loop(0, hb)
    def _(b):
        pltpu.sync_copy(ones_v, acc_s.at[sidx.at[b]], add=True)
        pltpu.sync_copy(ones_v, acc_d.at[didx.at[b]], add=True)

    plsc.subcore_barrier()
    pltpu.sync_copy(acc_s.at[pl.ds(s * rpt, rpt)],
                    out_hbm.at[c].at[0].at[pl.ds(s * rpt, rpt)])
    pltpu.sync_copy(acc_d.at[pl.ds(s * rpt, rpt)],
                    out_hbm.at[c].at[1].at[pl.ds(s * rpt, rpt)])


_deg_kernel = functools.partial(
    pl.kernel,
    out_type=jax.ShapeDtypeStruct((2, 2, _NROW, 16), _f32),
    mesh=_MESH,
    compiler_params=_SC_PARAMS,
    scratch_types=[
        pltpu.VMEM((_NB // 2, _K), _i32),
        pltpu.VMEM((_NB // 2, _K), _i32),
        pltpu.VMEM((_K, 16), _f32),
        pltpu.VMEM((_ROWS_PER_TILE, 16), _f32),
        pltpu.VMEM_SHARED((_NROW, 16), _f32),
        pltpu.VMEM_SHARED((_NROW, 16), _f32),
    ],
)(_deg_kernel_body)


def _agg512_body(p_hbm, srcp, dstp, zrow_hbm, out_hbm,
                 sidx, didx, b0, b1, b2, b3, b4, b5, b6, b7, acc,
                 g0, g1, g2, g3, g4, g5, g6, g7,
                 s0, s1, s2, s3, s4, s5, s6, s7):
    bufs = [b0, b1, b2, b3, b4, b5, b6, b7]
    gsems = [g0, g1, g2, g3, g4, g5, g6, g7]
    ssems = [s0, s1, s2, s3, s4, s5, s6, s7]
    c = lax.axis_index("c")
    s = lax.axis_index("s")
    rpt = _ROWS_PER_TILE
    pltpu.sync_copy(srcp.at[s], sidx)
    pltpu.sync_copy(dstp.at[s], didx)

    for ci in range(_NCH // 2):
        chunk = c * _NCH // 2 + ci
        tbl = p_hbm.at[chunk]
        och = out_hbm.at[chunk]

        pltpu.sync_copy(zrow_hbm, bufs[0])
        for j in range(rpt // _K):
            pltpu.sync_copy(bufs[0], acc.at[pl.ds(s * rpt + j * _K, _K)])
        plsc.subcore_barrier()

        def gstart(b, buf, sem):
            pltpu.async_copy(tbl.at[sidx.at[b]], buf, sem)

        def gwait(buf, sem):
            pltpu.make_async_copy(tbl.at[sidx.at[0]], buf, sem).wait()

        def sstart(b, buf, sem):
            pltpu.async_copy(buf, acc.at[didx.at[b]], sem, add=True)

        def swait(buf, sem):
            pltpu.make_async_copy(buf, acc.at[didx.at[0]], sem).wait()

        for k in range(_NBUF):
            gstart(k, bufs[k], gsems[k])

        @pl.loop(0, _NB, step=_NBUF)
        def _(b):
            for k in range(_NBUF):
                gwait(bufs[k], gsems[k])
                sstart(b + k, bufs[k], ssems[k])
            for k in range(_NBUF):
                swait(bufs[k], ssems[k])

                @pl.when(b + _NBUF + k < _NB)
                def _():
                    gstart(b + _NBUF + k, bufs[k], gsems[k])

        plsc.subcore_barrier()
        pltpu.sync_copy(acc.at[pl.ds(s * rpt, rpt)], och.at[pl.ds(s * rpt, rpt)])
        plsc.subcore_barrier()


_agg512 = functools.partial(
    pl.kernel,
    out_type=jax.ShapeDtypeStruct((_NCH, _NROW, _C), _f32),
    mesh=_MESH,
    compiler_params=_SC_PARAMS,
    scratch_types=(
        [pltpu.VMEM((_NB, _K), _i32)] * 2
        + [pltpu.VMEM((_K, _C), _f32)] * _NBUF
        + [pltpu.VMEM_SHARED((_NROW, _C), _f32)]
        + [pltpu.SemaphoreType.DMA] * (2 * _NBUF)
    ),
)(_agg512_body)


def _agg16_body(p_hbm, srcp, dstp, zrow_hbm, out_hbm,
                sidx, didx, buf0, buf1, acc, sg0, sg1, ss0, ss1):
    c = lax.axis_index("c")
    s = lax.axis_index("s")
    rpt = _ROWS_PER_TILE
    hb = _NB // 2
    pltpu.sync_copy(srcp.at[s].at[pl.ds(c * hb, hb)], sidx)
    pltpu.sync_copy(dstp.at[s].at[pl.ds(c * hb, hb)], didx)

    pltpu.sync_copy(zrow_hbm, buf0)
    for j in range(rpt // _K):
        pltpu.sync_copy(buf0, acc.at[pl.ds(s * rpt + j * _K, _K)])
    plsc.subcore_barrier()

    def gstart(b, buf, sem):
        pltpu.async_copy(p_hbm.at[sidx.at[b]], buf, sem)

    def gwait(buf, sem):
        pltpu.make_async_copy(p_hbm.at[sidx.at[0]], buf, sem).wait()

    def sstart(b, buf, sem):
        pltpu.async_copy(buf, acc.at[didx.at[b]], sem, add=True)

    def swait(buf, sem):
        pltpu.make_async_copy(buf, acc.at[didx.at[0]], sem).wait()

    gstart(0, buf0, sg0)

    @pl.loop(0, hb, step=2)
    def _(b):
        gwait(buf0, sg0)

        @pl.when(b > 0)
        def _():
            swait(buf1, ss1)

        gstart(b + 1, buf1, sg1)
        sstart(b, buf0, ss0)
        gwait(buf1, sg1)
        swait(buf0, ss0)

        @pl.when(b + 2 < hb)
        def _():
            gstart(b + 2, buf0, sg0)

        sstart(b + 1, buf1, ss1)

    swait(buf1, ss1)
    plsc.subcore_barrier()
    pltpu.sync_copy(acc.at[pl.ds(s * rpt, rpt)],
                    out_hbm.at[c].at[pl.ds(s * rpt, rpt)])


_agg16 = functools.partial(
    pl.kernel,
    out_type=jax.ShapeDtypeStruct((2, _NROW, 16), _f32),
    mesh=_MESH,
    compiler_params=_SC_PARAMS,
    scratch_types=[
        pltpu.VMEM((_NB // 2, _K), _i32),
        pltpu.VMEM((_NB // 2, _K), _i32),
        pltpu.VMEM((_K, 16), _f32),
        pltpu.VMEM((_K, 16), _f32),
        pltpu.VMEM_SHARED((_NROW, 16), _f32),
        pltpu.SemaphoreType.DMA,
        pltpu.SemaphoreType.DMA,
        pltpu.SemaphoreType.DMA,
        pltpu.SemaphoreType.DMA,
    ],
)(_agg16_body)


def _fused_in_body(x_ref, w_ref, b_ref, dg_ref, p1_ref, nin_ref, nout_ref):
    a = x_ref[0].astype(jnp.bfloat16)
    w = w_ref[0].astype(jnp.bfloat16)
    h = lax.dot_general(a, w, (((1,), (0,)), ((), ())),
                        preferred_element_type=_f32) + b_ref[0]
    dg = dg_ref[...]
    d_out = dg[0, 0, :, 0:1] + dg[1, 0, :, 0:1]
    d_in = dg[0, 1, :, 0:1] + dg[1, 1, :, 0:1]
    no = jnp.where(d_out > 0, lax.rsqrt(d_out), 0.0)
    ni = jnp.where(d_in > 0, lax.rsqrt(d_in), 0.0)
    blk = no.shape[0]
    nout_ref[...] = jnp.broadcast_to(no, (blk, _C))
    nin_ref[...] = jnp.broadcast_to(ni, (blk, _C))
    hb = h * no
    for cc in range(_NCH):
        p1_ref[cc] = hb[:, cc * _C:(cc + 1) * _C]


def _fused_in(xs, ws, bs, degp):
    blk = 1000
    return pl.pallas_call(
        _fused_in_body,
        grid=(2, 5),
        in_specs=[
            pl.BlockSpec((1, blk, 256), lambda j, i: (j, i, 0)),
            pl.BlockSpec((1, 256, 512), lambda j, i: (j, 0, 0)),
            pl.BlockSpec((1, 1, 512), lambda j, i: (j, 0, 0)),
            pl.BlockSpec((2, 2, blk, 16), lambda j, i: (0, 0, j * 5 + i, 0)),
        ],
        out_specs=[
            pl.BlockSpec((_NCH, blk, _C), lambda j, i: (0, j * 5 + i, 0)),
            pl.BlockSpec((blk, _C), lambda j, i: (j * 5 + i, 0)),
            pl.BlockSpec((blk, _C), lambda j, i: (j * 5 + i, 0)),
        ],
        out_shape=[
            jax.ShapeDtypeStruct((_NCH, _NROW, _C), _f32),
            jax.ShapeDtypeStruct((_NROW, _C), _f32),
            jax.ShapeDtypeStruct((_NROW, _C), _f32),
        ],
    )(xs, ws, bs, degp)


def _mid_body(a_ref, nin_ref, nout_ref, b_ref, w_ref, o_ref, *, nwo):
    nin = nin_ref[...]
    nout = nout_ref[...]
    blk = nin.shape[0]
    acc = jnp.zeros((blk, nwo), _f32)
    for cc in range(_NCH):
        hc = jnp.maximum(a_ref[cc] * nin + b_ref[cc], 0.0) * nout
        acc = acc + lax.dot_general(
            hc.astype(jnp.bfloat16), w_ref[cc].astype(jnp.bfloat16),
            (((1,), (0,)), ((), ())), preferred_element_type=_f32)
    if nwo == 512:
        for cc in range(_NCH):
            o_ref[cc] = acc[:, cc * _C:(cc + 1) * _C]
    else:
        o_ref[...] = acc


def _mid(a, nin, nout, b, w):
    blk = 1024
    nwo = w.shape[2]
    if nwo == 512:
        out_spec = pl.BlockSpec((_NCH, blk, _C), lambda i: (0, i, 0))
        out_shape = jax.ShapeDtypeStruct((_NCH, _NROW, _C), _f32)
    else:
        out_spec = pl.BlockSpec((blk, nwo), lambda i: (i, 0))
        out_shape = jax.ShapeDtypeStruct((_NROW, nwo), _f32)
    return pl.pallas_call(
        functools.partial(_mid_body, nwo=nwo),
        grid=(_NROW // blk,),
        in_specs=[
            pl.BlockSpec((_NCH, blk, _C), lambda i: (0, i, 0)),
            pl.BlockSpec((blk, _C), lambda i: (i, 0)),
            pl.BlockSpec((blk, _C), lambda i: (i, 0)),
            pl.BlockSpec((_NCH, 1, _C), lambda i: (0, 0, 0)),
            pl.BlockSpec((_NCH, _C, nwo), lambda i: (0, 0, 0)),
        ],
        out_specs=out_spec,
        out_shape=out_shape,
    )(a, nin, nout, b, w)


def _final_body(a_ref, nin_ref, b_ref, o_ref):
    a = a_ref[0] + a_ref[1]
    o_ref[...] = a * nin_ref[:, :16] + b_ref[...]


def _final(a3p, nin, b2):
    blk = 1000
    return pl.pallas_call(
        _final_body,
        grid=(_N // blk,),
        in_specs=[
            pl.BlockSpec((2, blk, 16), lambda i: (0, i, 0)),
            pl.BlockSpec((blk, _C), lambda i: (i, 0)),
            pl.BlockSpec((1, 16), lambda i: (0, 0)),
        ],
        out_specs=pl.BlockSpec((blk, 16), lambda i: (i, 0)),
        out_shape=jax.ShapeDtypeStruct((_N, 16), _f32),
    )(a3p, nin, b2)


def kernel(x0, x1, edge_index, fc0_w, fc0_b, fc1_w, fc1_b, conv0_b, conv1_w,
           conv1_b, conv2_w, conv2_b):
    src = edge_index[0]
    dst = edge_index[1]
    pad = jnp.full((_EPAD - _E,), _N, _i32)
    srcp = jnp.concatenate([src, pad]).reshape(_NSUB, _NB, _K)
    dstp = jnp.concatenate([dst, pad]).reshape(_NSUB, _NB, _K)

    ones128 = jnp.ones((_K, 16), _f32)
    zrow1 = jnp.zeros((_ROWS_PER_TILE, 16), _f32)
    zrow128 = jnp.zeros((_K, _C), _f32)
    zrow16 = jnp.zeros((_K, 16), _f32)

    xs = jnp.stack([x0, x1])
    ws = jnp.stack([fc0_w, fc1_w])
    bs = jnp.stack([fc0_b, fc1_b]).reshape(2, 1, 512)

    degp = _deg_kernel(srcp, dstp, ones128, zrow1)
    p1, nin, nout = _fused_in(xs, ws, bs, degp)

    a1 = _agg512(p1, srcp, dstp, zrow128)
    p2 = _mid(a1, nin, nout, conv0_b.reshape(_NCH, 1, _C),
              conv1_w.reshape(_NCH, _C, 512))
    a2 = _agg512(p2, srcp, dstp, zrow128)
    p3 = _mid(a2, nin, nout, conv1_b.reshape(_NCH, 1, _C),
              conv2_w.reshape(_NCH, _C, 16))
    a3p = _agg16(p3, srcp, dstp, zrow16)
    out = _final(a3p, nin, conv2_b.reshape(1, 16))
    return out
